# Initial kernel scaffold; baseline (speedup 1.0000x reference)
import jax
import jax.numpy as jnp
from jax.experimental import pallas as pl


def kernel(x, edge_index, edge_attr, We, be, Wm, bm, Wp1, bp1, Wp2, bp2):
    raise NotImplementedError("write your pallas kernel here")


# speedup vs baseline: 2.3820x; 2.3820x over previous
import functools

import jax
import jax.numpy as jnp
from jax import lax
from jax.experimental import pallas as pl
from jax.experimental.pallas import tpu as pltpu
from jax.experimental.pallas import tpu_sc as plsc

N, E, D, ED = 10000, 320000, 128, 3
NC, NS, LANES = 2, 16, 16
NW = NC * NS
EW = E // NW
CHUNK = 80
NCHUNKS = EW // CHUNK
ROWS_PER_TILE = 624
TAIL_ROWS = N - NS * ROWS_PER_TILE
DC = D // LANES


def _sc_body(src_hbm, dst_hbm, a0_hbm, a1_hbm, a2_hbm, we_hbm, be_hbm,
             zeros_hbm, x_hbm, out_hbm, src_idx, dst_idx, a0_b, a1_b, a2_b,
             xbuf, wbuf, bbuf, acc, sem):
    cid = lax.axis_index("c")
    sid = lax.axis_index("s")
    wid = sid * NC + cid

    pltpu.sync_copy(we_hbm, wbuf)
    pltpu.sync_copy(be_hbm, bbuf)
    pltpu.sync_copy(zeros_hbm.at[pl.ds(sid * ROWS_PER_TILE, ROWS_PER_TILE)],
                    acc.at[pl.ds(sid * ROWS_PER_TILE, ROWS_PER_TILE)])

    @pl.when(sid == NS - 1)
    def _zero_tail():
        pltpu.sync_copy(zeros_hbm.at[pl.ds(NS * ROWS_PER_TILE, TAIL_ROWS)],
                        acc.at[pl.ds(NS * ROWS_PER_TILE, TAIL_ROWS)])
    wev = [[wbuf[pl.ds(k * D + c * LANES, LANES)] for c in range(DC)]
           for k in range(ED)]
    bev = [bbuf[pl.ds(c * LANES, LANES)] for c in range(DC)]
    plsc.subcore_barrier()

    base = wid * EW

    def chunk_body(i, carry):
        off = base + i * CHUNK
        pltpu.sync_copy(src_hbm.at[pl.ds(off, CHUNK)], src_idx)
        pltpu.sync_copy(dst_hbm.at[pl.ds(off, CHUNK)], dst_idx)
        pltpu.sync_copy(a0_hbm.at[pl.ds(off, CHUNK)], a0_b)
        pltpu.sync_copy(a1_hbm.at[pl.ds(off, CHUNK)], a1_b)
        pltpu.sync_copy(a2_hbm.at[pl.ds(off, CHUNK)], a2_b)
        pltpu.async_copy(x_hbm.at[src_idx], xbuf, sem).wait()

        def group_body(g, c2):
            a0v = a0_b[pl.ds(g * LANES, LANES)]
            a1v = a1_b[pl.ds(g * LANES, LANES)]
            a2v = a2_b[pl.ds(g * LANES, LANES)]
            for lane in range(LANES):
                e = g * LANES + lane
                a0 = a0v[lane]
                a1 = a1v[lane]
                a2 = a2v[lane]
                for c in range(DC):
                    xv = xbuf[e, pl.ds(c * LANES, LANES)]
                    v = (xv + bev[c] + a0 * wev[0][c] + a1 * wev[1][c]
                         + a2 * wev[2][c])
                    xbuf[e, pl.ds(c * LANES, LANES)] = jnp.maximum(v, 0.0)
            return c2

        lax.fori_loop(0, CHUNK // LANES, group_body, 0)
        pltpu.sync_copy(xbuf, acc.at[dst_idx], add=True)
        return carry

    lax.fori_loop(0, NCHUNKS, chunk_body, 0)
    plsc.subcore_barrier()
    pltpu.sync_copy(acc.at[pl.ds(sid * ROWS_PER_TILE, ROWS_PER_TILE)],
                    out_hbm.at[cid, pl.ds(sid * ROWS_PER_TILE, ROWS_PER_TILE)])

    @pl.when(sid == NS - 1)
    def _write_tail():
        pltpu.sync_copy(acc.at[pl.ds(NS * ROWS_PER_TILE, TAIL_ROWS)],
                        out_hbm.at[cid, pl.ds(NS * ROWS_PER_TILE, TAIL_ROWS)])


_sc_layer = functools.partial(
    pl.kernel,
    out_type=jax.ShapeDtypeStruct((NC, N, D), jnp.float32),
    mesh=plsc.VectorSubcoreMesh(core_axis_name="c", subcore_axis_name="s"),
    scratch_types=[
        pltpu.VMEM((CHUNK,), jnp.int32),
        pltpu.VMEM((CHUNK,), jnp.int32),
        pltpu.VMEM((CHUNK,), jnp.float32),
        pltpu.VMEM((CHUNK,), jnp.float32),
        pltpu.VMEM((CHUNK,), jnp.float32),
        pltpu.VMEM((CHUNK, D), jnp.float32),
        pltpu.VMEM((ED * D,), jnp.float32),
        pltpu.VMEM((D,), jnp.float32),
        pltpu.VMEM_SHARED((N, D), jnp.float32),
        pltpu.SemaphoreType.DMA,
    ],
)(_sc_body)


def _tc_body_mid(x_ref, acc_ref, wm_ref, bm_ref, o_ref):
    h = x_ref[...] + acc_ref[0] + acc_ref[1]
    h = jnp.maximum(jnp.dot(h, wm_ref[0], preferred_element_type=jnp.float32)
                    + bm_ref[0], 0.0)
    h = jnp.maximum(jnp.dot(h, wm_ref[1], preferred_element_type=jnp.float32)
                    + bm_ref[1], 0.0)
    h = jnp.dot(h, wm_ref[2], preferred_element_type=jnp.float32) + bm_ref[2]
    o_ref[...] = jnp.maximum(h, 0.0) + x_ref[...]


def _tc_body_final(x_ref, acc_ref, wm_ref, bm_ref, wp1_ref, bp1_ref, wp2_ref,
                   bp2_ref, o_ref):
    h = x_ref[...] + acc_ref[0] + acc_ref[1]
    h = jnp.maximum(jnp.dot(h, wm_ref[0], preferred_element_type=jnp.float32)
                    + bm_ref[0], 0.0)
    h = jnp.maximum(jnp.dot(h, wm_ref[1], preferred_element_type=jnp.float32)
                    + bm_ref[1], 0.0)
    h = jnp.dot(h, wm_ref[2], preferred_element_type=jnp.float32) + bm_ref[2]
    h = jnp.maximum(h, 0.0) + x_ref[...]
    h = jnp.maximum(jnp.dot(h, wp1_ref[...], preferred_element_type=jnp.float32)
                    + bp1_ref[...], 0.0)
    o_ref[...] = (jnp.dot(h, wp2_ref[...], preferred_element_type=jnp.float32)
                  + bp2_ref[...])


_TC_BLK = 1000
_TC_GRID = N // _TC_BLK

_x_spec = pl.BlockSpec((_TC_BLK, D), lambda i: (i, 0))
_acc_spec = pl.BlockSpec((NC, _TC_BLK, D), lambda i: (0, i, 0))
_wm_spec = pl.BlockSpec((3, D, D), lambda i: (0, 0, 0))
_bm_spec = pl.BlockSpec((3, D), lambda i: (0, 0))
_w_spec = pl.BlockSpec((D, D), lambda i: (0, 0))
_b_spec = pl.BlockSpec((1, D), lambda i: (0, 0))

_tc_mid = pl.pallas_call(
    _tc_body_mid,
    grid=(_TC_GRID,),
    in_specs=[_x_spec, _acc_spec, _wm_spec, _bm_spec],
    out_specs=_x_spec,
    out_shape=jax.ShapeDtypeStruct((N, D), jnp.float32),
)

_tc_final = pl.pallas_call(
    _tc_body_final,
    grid=(_TC_GRID,),
    in_specs=[_x_spec, _acc_spec, _wm_spec, _bm_spec,
              _w_spec, _b_spec, _w_spec, _b_spec],
    out_specs=_x_spec,
    out_shape=jax.ShapeDtypeStruct((N, D), jnp.float32),
)


def kernel(x, edge_index, edge_attr, We, be, Wm, bm, Wp1, bp1, Wp2, bp2):
    src = edge_index[0]
    dst = edge_index[1]
    zeros_nd = jnp.zeros((N, D), jnp.float32)
    bp1r = bp1.reshape(1, D)
    bp2r = bp2.reshape(1, D)
    a0 = edge_attr[:, 0]
    a1 = edge_attr[:, 1]
    a2 = edge_attr[:, 2]
    for l in range(3):
        agg = _sc_layer(src, dst, a0, a1, a2, We[l].reshape(-1), be[l],
                        zeros_nd, x)
        if l < 2:
            x = _tc_mid(x, agg, Wm[l], bm[l])
        else:
            x = _tc_final(x, agg, Wm[l], bm[l], Wp1, bp1r, Wp2, bp2r)
    return x


# speedup vs baseline: 2.7129x; 1.1389x over previous
import functools

import jax
import jax.numpy as jnp
from jax import lax
from jax.experimental import pallas as pl
from jax.experimental.pallas import tpu as pltpu
from jax.experimental.pallas import tpu_sc as plsc

N, E, D, ED = 10000, 320000, 128, 3
NC, NS, LANES = 2, 16, 16
NW = NC * NS
EW = E // NW
CHUNK = 40
NCHUNKS = EW // CHUNK
NBUF = 5
AW = 8
ROWS_PER_TILE = 624
TAIL_ROWS = N - NS * ROWS_PER_TILE
DC = D // LANES


def _sc_body(src_hbm, dst_hbm, attr_hbm, we_hbm, be_hbm, zeros_hbm, x_hbm,
             out_hbm, *scr):
    srcb = scr[0:NBUF]
    dstb = scr[NBUF:2 * NBUF]
    atb = scr[2 * NBUF:3 * NBUF]
    xb = scr[3 * NBUF:4 * NBUF]
    wbuf, bbuf, acc = scr[4 * NBUF:4 * NBUF + 3]
    msem = scr[4 * NBUF + 3:5 * NBUF + 3]
    gsem = scr[5 * NBUF + 3:6 * NBUF + 3]
    ssem = scr[6 * NBUF + 3:7 * NBUF + 3]

    cid = lax.axis_index("c")
    sid = lax.axis_index("s")
    wid = sid * NC + cid
    base = wid * EW

    pltpu.sync_copy(we_hbm, wbuf)
    pltpu.sync_copy(be_hbm, bbuf)
    pltpu.sync_copy(zeros_hbm.at[pl.ds(sid * ROWS_PER_TILE, ROWS_PER_TILE)],
                    acc.at[pl.ds(sid * ROWS_PER_TILE, ROWS_PER_TILE)])

    @pl.when(sid == NS - 1)
    def _zero_tail():
        pltpu.sync_copy(zeros_hbm.at[pl.ds(NS * ROWS_PER_TILE, TAIL_ROWS)],
                        acc.at[pl.ds(NS * ROWS_PER_TILE, TAIL_ROWS)])

    wev = [[wbuf[pl.ds(k * D + c * LANES, LANES)] for c in range(DC)]
           for k in range(ED)]
    bev = [bbuf[pl.ds(c * LANES, LANES)] for c in range(DC)]
    plsc.subcore_barrier()

    def meta_descs(c, b):
        off = base + c * CHUNK
        return (
            pltpu.make_async_copy(src_hbm.at[pl.ds(off, CHUNK)], srcb[b],
                                  msem[b]),
            pltpu.make_async_copy(dst_hbm.at[pl.ds(off, CHUNK)], dstb[b],
                                  msem[b]),
            pltpu.make_async_copy(attr_hbm.at[pl.ds(off * AW, CHUNK * AW)],
                                  atb[b].at[pl.ds(0, CHUNK * AW)], msem[b]),
        )

    def issue_meta(c, b):
        for d in meta_descs(c, b):
            d.start()

    def wait_meta(c, b):
        for d in meta_descs(c, b):
            d.wait()

    def gather_desc(b):
        return pltpu.make_async_copy(x_hbm.at[srcb[b]], xb[b], gsem[b])

    def scatter_desc(b):
        return pltpu.make_async_copy(xb[b], acc.at[dstb[b]], ssem[b])

    def compute(b):
        xbuf = xb[b]
        abuf = atb[b]

        def edge_body(e, carry):
            av = abuf[pl.ds(e * AW, LANES)]
            a0 = av[0]
            a1 = av[1]
            a2 = av[2]
            for c in range(DC):
                xv = xbuf[e, pl.ds(c * LANES, LANES)]
                v = (xv + bev[c] + a0 * wev[0][c] + a1 * wev[1][c]
                     + a2 * wev[2][c])
                xbuf[e, pl.ds(c * LANES, LANES)] = jnp.maximum(v, 0.0)
            return carry

        lax.fori_loop(0, CHUNK, edge_body, 0, unroll=2)

    def slot(s, k):
        gather_desc(k).wait()

        @pl.when(s >= 2)
        def _drain_scatter():
            scatter_desc((k - 2) % NBUF).wait()

        @pl.when(s + 2 <= NCHUNKS - 1)
        def _prefetch_meta():
            issue_meta(s + 2, (k + 2) % NBUF)

        @pl.when(s + 1 <= NCHUNKS - 1)
        def _launch_gather():
            wait_meta(s + 1, (k + 1) % NBUF)
            pltpu.async_copy(x_hbm.at[srcb[(k + 1) % NBUF]],
                             xb[(k + 1) % NBUF], gsem[(k + 1) % NBUF])

        compute(k)
        pltpu.async_copy(xb[k], acc.at[dstb[k]], ssem[k], add=True)

    issue_meta(0, 0)
    issue_meta(1, 1)
    wait_meta(0, 0)
    pltpu.async_copy(x_hbm.at[srcb[0]], xb[0], gsem[0])

    def outer(t, carry):
        for k in range(NBUF):
            slot(t * NBUF + k, k)
        return carry

    lax.fori_loop(0, NCHUNKS // NBUF, outer, 0)

    scatter_desc((NCHUNKS - 2) % NBUF).wait()
    scatter_desc((NCHUNKS - 1) % NBUF).wait()

    plsc.subcore_barrier()
    pltpu.sync_copy(acc.at[pl.ds(sid * ROWS_PER_TILE, ROWS_PER_TILE)],
                    out_hbm.at[cid, pl.ds(sid * ROWS_PER_TILE, ROWS_PER_TILE)])

    @pl.when(sid == NS - 1)
    def _write_tail():
        pltpu.sync_copy(acc.at[pl.ds(NS * ROWS_PER_TILE, TAIL_ROWS)],
                        out_hbm.at[cid, pl.ds(NS * ROWS_PER_TILE, TAIL_ROWS)])


_sc_layer = functools.partial(
    pl.kernel,
    out_type=jax.ShapeDtypeStruct((NC, N, D), jnp.float32),
    mesh=plsc.VectorSubcoreMesh(core_axis_name="c", subcore_axis_name="s"),
    scratch_types=(
        [pltpu.VMEM((CHUNK,), jnp.int32) for _ in range(NBUF)]
        + [pltpu.VMEM((CHUNK,), jnp.int32) for _ in range(NBUF)]
        + [pltpu.VMEM((CHUNK * AW + LANES,), jnp.float32)
           for _ in range(NBUF)]
        + [pltpu.VMEM((CHUNK, D), jnp.float32) for _ in range(NBUF)]
        + [
            pltpu.VMEM((ED * D,), jnp.float32),
            pltpu.VMEM((D,), jnp.float32),
            pltpu.VMEM_SHARED((N, D), jnp.float32),
        ]
        + [pltpu.SemaphoreType.DMA for _ in range(3 * NBUF)]
    ),
)(_sc_body)


def _tc_body_mid(x_ref, acc_ref, wm_ref, bm_ref, o_ref):
    h = x_ref[...] + acc_ref[0] + acc_ref[1]
    h = jnp.maximum(jnp.dot(h, wm_ref[0], preferred_element_type=jnp.float32)
                    + bm_ref[0], 0.0)
    h = jnp.maximum(jnp.dot(h, wm_ref[1], preferred_element_type=jnp.float32)
                    + bm_ref[1], 0.0)
    h = jnp.dot(h, wm_ref[2], preferred_element_type=jnp.float32) + bm_ref[2]
    o_ref[...] = jnp.maximum(h, 0.0) + x_ref[...]


def _tc_body_final(x_ref, acc_ref, wm_ref, bm_ref, wp1_ref, bp1_ref, wp2_ref,
                   bp2_ref, o_ref):
    h = x_ref[...] + acc_ref[0] + acc_ref[1]
    h = jnp.maximum(jnp.dot(h, wm_ref[0], preferred_element_type=jnp.float32)
                    + bm_ref[0], 0.0)
    h = jnp.maximum(jnp.dot(h, wm_ref[1], preferred_element_type=jnp.float32)
                    + bm_ref[1], 0.0)
    h = jnp.dot(h, wm_ref[2], preferred_element_type=jnp.float32) + bm_ref[2]
    h = jnp.maximum(h, 0.0) + x_ref[...]
    h = jnp.maximum(jnp.dot(h, wp1_ref[...], preferred_element_type=jnp.float32)
                    + bp1_ref[...], 0.0)
    o_ref[...] = (jnp.dot(h, wp2_ref[...], preferred_element_type=jnp.float32)
                  + bp2_ref[...])


_TC_BLK = 1000
_TC_GRID = N // _TC_BLK

_x_spec = pl.BlockSpec((_TC_BLK, D), lambda i: (i, 0))
_acc_spec = pl.BlockSpec((NC, _TC_BLK, D), lambda i: (0, i, 0))
_wm_spec = pl.BlockSpec((3, D, D), lambda i: (0, 0, 0))
_bm_spec = pl.BlockSpec((3, D), lambda i: (0, 0))
_w_spec = pl.BlockSpec((D, D), lambda i: (0, 0))
_b_spec = pl.BlockSpec((1, D), lambda i: (0, 0))

_tc_mid = pl.pallas_call(
    _tc_body_mid,
    grid=(_TC_GRID,),
    in_specs=[_x_spec, _acc_spec, _wm_spec, _bm_spec],
    out_specs=_x_spec,
    out_shape=jax.ShapeDtypeStruct((N, D), jnp.float32),
)

_tc_final = pl.pallas_call(
    _tc_body_final,
    grid=(_TC_GRID,),
    in_specs=[_x_spec, _acc_spec, _wm_spec, _bm_spec,
              _w_spec, _b_spec, _w_spec, _b_spec],
    out_specs=_x_spec,
    out_shape=jax.ShapeDtypeStruct((N, D), jnp.float32),
)


def kernel(x, edge_index, edge_attr, We, be, Wm, bm, Wp1, bp1, Wp2, bp2):
    src = edge_index[0]
    dst = edge_index[1]
    attr8 = jnp.pad(edge_attr, ((0, 0), (0, AW - ED))).reshape(-1)
    zeros_nd = jnp.zeros((N, D), jnp.float32)
    bp1r = bp1.reshape(1, D)
    bp2r = bp2.reshape(1, D)
    for l in range(3):
        agg = _sc_layer(src, dst, attr8, We[l].reshape(-1), be[l],
                        zeros_nd, x)
        if l < 2:
            x = _tc_mid(x, agg, Wm[l], bm[l])
        else:
            x = _tc_final(x, agg, Wm[l], bm[l], Wp1, bp1r, Wp2, bp2r)
    return x


# speedup vs baseline: 4.1469x; 1.5286x over previous
import functools

import jax
import jax.numpy as jnp
from jax import lax
from jax.experimental import pallas as pl
from jax.experimental.pallas import tpu as pltpu
from jax.experimental.pallas import tpu_sc as plsc

N, E, D, ED = 10000, 320000, 128, 3
NC, NS, LANES = 2, 16, 16
NW = NC * NS
EW = E // NW
CHUNK = 40
NCHUNKS = EW // CHUNK
NBUF = 5
ROWS_PER_TILE = 624
TAIL_ROWS = N - NS * ROWS_PER_TILE
DC = D // LANES


def _sc_body(src_hbm, dst_hbm, el_hbm, zeros_hbm, x_hbm, out_hbm, *scr):
    srcb = scr[0:NBUF]
    dstb = scr[NBUF:2 * NBUF]
    eb = scr[2 * NBUF:3 * NBUF]
    acc = scr[3 * NBUF]
    msem = scr[3 * NBUF + 1:4 * NBUF + 1]
    gsem = scr[4 * NBUF + 1:5 * NBUF + 1]
    ssem = scr[5 * NBUF + 1:6 * NBUF + 1]

    cid = lax.axis_index("c")
    sid = lax.axis_index("s")
    wid = sid * NC + cid
    base = wid * EW

    pltpu.sync_copy(zeros_hbm.at[pl.ds(sid * ROWS_PER_TILE, ROWS_PER_TILE)],
                    acc.at[pl.ds(sid * ROWS_PER_TILE, ROWS_PER_TILE)])

    @pl.when(sid == NS - 1)
    def _zero_tail():
        pltpu.sync_copy(zeros_hbm.at[pl.ds(NS * ROWS_PER_TILE, TAIL_ROWS)],
                        acc.at[pl.ds(NS * ROWS_PER_TILE, TAIL_ROWS)])

    plsc.subcore_barrier()

    def meta_descs(c, b):
        off = base + c * CHUNK
        return (
            pltpu.make_async_copy(src_hbm.at[pl.ds(off, CHUNK)], srcb[b],
                                  msem[b]),
            pltpu.make_async_copy(dst_hbm.at[pl.ds(off, CHUNK)], dstb[b],
                                  msem[b]),
            pltpu.make_async_copy(el_hbm.at[pl.ds(off, CHUNK)], eb[b],
                                  msem[b]),
        )

    def issue_meta(c, b):
        for d in meta_descs(c, b):
            d.start()

    def wait_meta(c, b):
        for d in meta_descs(c, b):
            d.wait()

    def gather_desc(b):
        return pltpu.make_async_copy(x_hbm.at[srcb[b]], eb[b], gsem[b])

    def scatter_desc(b):
        return pltpu.make_async_copy(eb[b], acc.at[dstb[b]], ssem[b])

    def compute(b):
        ebuf = eb[b]

        @plsc.parallel_loop(0, CHUNK, 1, unroll=4)
        def edge_body(e):
            for c in range(DC):
                v = ebuf[e, pl.ds(c * LANES, LANES)]
                ebuf[e, pl.ds(c * LANES, LANES)] = jnp.maximum(v, 0.0)

    def slot(s, k):
        gather_desc(k).wait()

        @pl.when(s >= 2)
        def _drain_scatter():
            scatter_desc((k - 2) % NBUF).wait()

        @pl.when(s + 2 <= NCHUNKS - 1)
        def _prefetch_meta():
            issue_meta(s + 2, (k + 2) % NBUF)

        @pl.when(s + 1 <= NCHUNKS - 1)
        def _launch_gather():
            wait_meta(s + 1, (k + 1) % NBUF)
            pltpu.async_copy(x_hbm.at[srcb[(k + 1) % NBUF]],
                             eb[(k + 1) % NBUF], gsem[(k + 1) % NBUF],
                             add=True)

        compute(k)
        pltpu.async_copy(eb[k], acc.at[dstb[k]], ssem[k], add=True)

    issue_meta(0, 0)
    issue_meta(1, 1)
    wait_meta(0, 0)
    pltpu.async_copy(x_hbm.at[srcb[0]], eb[0], gsem[0], add=True)

    def outer(t, carry):
        for k in range(NBUF):
            slot(t * NBUF + k, k)
        return carry

    lax.fori_loop(0, NCHUNKS // NBUF, outer, 0)

    scatter_desc((NCHUNKS - 2) % NBUF).wait()
    scatter_desc((NCHUNKS - 1) % NBUF).wait()

    plsc.subcore_barrier()
    pltpu.sync_copy(acc.at[pl.ds(sid * ROWS_PER_TILE, ROWS_PER_TILE)],
                    out_hbm.at[cid, pl.ds(sid * ROWS_PER_TILE, ROWS_PER_TILE)])

    @pl.when(sid == NS - 1)
    def _write_tail():
        pltpu.sync_copy(acc.at[pl.ds(NS * ROWS_PER_TILE, TAIL_ROWS)],
                        out_hbm.at[cid, pl.ds(NS * ROWS_PER_TILE, TAIL_ROWS)])


_sc_layer = functools.partial(
    pl.kernel,
    out_type=jax.ShapeDtypeStruct((NC, N, D), jnp.float32),
    mesh=plsc.VectorSubcoreMesh(core_axis_name="c", subcore_axis_name="s"),
    scratch_types=(
        [pltpu.VMEM((CHUNK,), jnp.int32) for _ in range(NBUF)]
        + [pltpu.VMEM((CHUNK,), jnp.int32) for _ in range(NBUF)]
        + [pltpu.VMEM((CHUNK, D), jnp.float32) for _ in range(NBUF)]
        + [pltpu.VMEM_SHARED((N, D), jnp.float32)]
        + [pltpu.SemaphoreType.DMA for _ in range(3 * NBUF)]
    ),
)(_sc_body)


_EBLK = 8000


def _tc_edge_body(attr_ref, we_ref, be_ref, o_ref):
    o_ref[...] = (jnp.dot(attr_ref[...], we_ref[...],
                          preferred_element_type=jnp.float32) + be_ref[...])


_tc_edge = pl.pallas_call(
    _tc_edge_body,
    grid=(E // _EBLK,),
    in_specs=[pl.BlockSpec((_EBLK, ED), lambda i: (i, 0)),
              pl.BlockSpec((ED, D), lambda i: (0, 0)),
              pl.BlockSpec((1, D), lambda i: (0, 0))],
    out_specs=pl.BlockSpec((_EBLK, D), lambda i: (i, 0)),
    out_shape=jax.ShapeDtypeStruct((E, D), jnp.float32),
)


def _tc_body_mid(x_ref, acc_ref, wm_ref, bm_ref, o_ref):
    h = x_ref[...] + acc_ref[0] + acc_ref[1]
    h = jnp.maximum(jnp.dot(h, wm_ref[0], preferred_element_type=jnp.float32)
                    + bm_ref[0], 0.0)
    h = jnp.maximum(jnp.dot(h, wm_ref[1], preferred_element_type=jnp.float32)
                    + bm_ref[1], 0.0)
    h = jnp.dot(h, wm_ref[2], preferred_element_type=jnp.float32) + bm_ref[2]
    o_ref[...] = jnp.maximum(h, 0.0) + x_ref[...]


def _tc_body_final(x_ref, acc_ref, wm_ref, bm_ref, wp1_ref, bp1_ref, wp2_ref,
                   bp2_ref, o_ref):
    h = x_ref[...] + acc_ref[0] + acc_ref[1]
    h = jnp.maximum(jnp.dot(h, wm_ref[0], preferred_element_type=jnp.float32)
                    + bm_ref[0], 0.0)
    h = jnp.maximum(jnp.dot(h, wm_ref[1], preferred_element_type=jnp.float32)
                    + bm_ref[1], 0.0)
    h = jnp.dot(h, wm_ref[2], preferred_element_type=jnp.float32) + bm_ref[2]
    h = jnp.maximum(h, 0.0) + x_ref[...]
    h = jnp.maximum(jnp.dot(h, wp1_ref[...], preferred_element_type=jnp.float32)
                    + bp1_ref[...], 0.0)
    o_ref[...] = (jnp.dot(h, wp2_ref[...], preferred_element_type=jnp.float32)
                  + bp2_ref[...])


_TC_BLK = 1000
_TC_GRID = N // _TC_BLK

_x_spec = pl.BlockSpec((_TC_BLK, D), lambda i: (i, 0))
_acc_spec = pl.BlockSpec((NC, _TC_BLK, D), lambda i: (0, i, 0))
_wm_spec = pl.BlockSpec((3, D, D), lambda i: (0, 0, 0))
_bm_spec = pl.BlockSpec((3, D), lambda i: (0, 0))
_w_spec = pl.BlockSpec((D, D), lambda i: (0, 0))
_b_spec = pl.BlockSpec((1, D), lambda i: (0, 0))

_tc_mid = pl.pallas_call(
    _tc_body_mid,
    grid=(_TC_GRID,),
    in_specs=[_x_spec, _acc_spec, _wm_spec, _bm_spec],
    out_specs=_x_spec,
    out_shape=jax.ShapeDtypeStruct((N, D), jnp.float32),
)

_tc_final = pl.pallas_call(
    _tc_body_final,
    grid=(_TC_GRID,),
    in_specs=[_x_spec, _acc_spec, _wm_spec, _bm_spec,
              _w_spec, _b_spec, _w_spec, _b_spec],
    out_specs=_x_spec,
    out_shape=jax.ShapeDtypeStruct((N, D), jnp.float32),
)


def kernel(x, edge_index, edge_attr, We, be, Wm, bm, Wp1, bp1, Wp2, bp2):
    src = edge_index[0]
    dst = edge_index[1]
    zeros_nd = jnp.zeros((N, D), jnp.float32)
    bp1r = bp1.reshape(1, D)
    bp2r = bp2.reshape(1, D)
    els = [_tc_edge(edge_attr, We[l], be[l].reshape(1, D)) for l in range(3)]
    for l in range(3):
        agg = _sc_layer(src, dst, els[l], zeros_nd, x)
        if l < 2:
            x = _tc_mid(x, agg, Wm[l], bm[l])
        else:
            x = _tc_final(x, agg, Wm[l], bm[l], Wp1, bp1r, Wp2, bp2r)
    return x


# speedup vs baseline: 4.1644x; 1.0042x over previous
import functools

import jax
import jax.numpy as jnp
from jax import lax
from jax.experimental import pallas as pl
from jax.experimental.pallas import tpu as pltpu
from jax.experimental.pallas import tpu_sc as plsc

N, E, D, ED = 10000, 320000, 128, 3
NC, NS, LANES = 2, 16, 16
NW = NC * NS
EW = E // NW
CHUNK = 40
NCHUNKS = EW // CHUNK
NBUF = 6
NSLOTS = ((NCHUNKS + NBUF - 1) // NBUF) * NBUF
ROWS_PER_TILE = 624
TAIL_ROWS = N - NS * ROWS_PER_TILE
DC = D // LANES


def _sc_body(src_hbm, dst_hbm, el_hbm, zeros_hbm, x_hbm, out_hbm, *scr):
    srcb = scr[0:NBUF]
    dstb = scr[NBUF:2 * NBUF]
    eb = scr[2 * NBUF:3 * NBUF]
    acc = scr[3 * NBUF]
    msem = scr[3 * NBUF + 1:4 * NBUF + 1]
    gsem = scr[4 * NBUF + 1:5 * NBUF + 1]
    ssem = scr[5 * NBUF + 1:6 * NBUF + 1]

    cid = lax.axis_index("c")
    sid = lax.axis_index("s")
    wid = sid * NC + cid
    base = wid * EW

    pltpu.sync_copy(zeros_hbm.at[pl.ds(sid * ROWS_PER_TILE, ROWS_PER_TILE)],
                    acc.at[pl.ds(sid * ROWS_PER_TILE, ROWS_PER_TILE)])

    @pl.when(sid == NS - 1)
    def _zero_tail():
        pltpu.sync_copy(zeros_hbm.at[pl.ds(NS * ROWS_PER_TILE, TAIL_ROWS)],
                        acc.at[pl.ds(NS * ROWS_PER_TILE, TAIL_ROWS)])

    plsc.subcore_barrier()

    def meta_descs(c, b):
        off = base + c * CHUNK
        return (
            pltpu.make_async_copy(src_hbm.at[pl.ds(off, CHUNK)], srcb[b],
                                  msem[b]),
            pltpu.make_async_copy(dst_hbm.at[pl.ds(off, CHUNK)], dstb[b],
                                  msem[b]),
            pltpu.make_async_copy(el_hbm.at[pl.ds(off, CHUNK)], eb[b],
                                  msem[b]),
        )

    def issue_meta(c, b):
        for d in meta_descs(c, b):
            d.start()

    def wait_meta(c, b):
        for d in meta_descs(c, b):
            d.wait()

    def gather_desc(b):
        return pltpu.make_async_copy(x_hbm.at[srcb[b]], eb[b], gsem[b])

    def scatter_desc(b):
        return pltpu.make_async_copy(eb[b], acc.at[dstb[b]], ssem[b])

    def compute(b):
        ebuf = eb[b]

        @plsc.parallel_loop(0, CHUNK, 1, unroll=4)
        def edge_body(e):
            for c in range(DC):
                v = ebuf[e, pl.ds(c * LANES, LANES)]
                ebuf[e, pl.ds(c * LANES, LANES)] = jnp.maximum(v, 0.0)

    def slot(s, k):
        @pl.when(s <= NCHUNKS - 1)
        def _wait_gather():
            gather_desc(k).wait()

        @pl.when(s >= 2)
        def _drain_scatter():
            scatter_desc((k - 2) % NBUF).wait()

        @pl.when(s + 3 <= NCHUNKS - 1)
        def _prefetch_meta():
            issue_meta(s + 3, (k + 3) % NBUF)

        @pl.when(s + 1 <= NCHUNKS - 1)
        def _launch_gather():
            wait_meta(s + 1, (k + 1) % NBUF)
            pltpu.async_copy(x_hbm.at[srcb[(k + 1) % NBUF]],
                             eb[(k + 1) % NBUF], gsem[(k + 1) % NBUF],
                             add=True)

        @pl.when(s <= NCHUNKS - 1)
        def _compute_scatter():
            compute(k)
            pltpu.async_copy(eb[k], acc.at[dstb[k]], ssem[k], add=True)

    issue_meta(0, 0)
    issue_meta(1, 1)
    issue_meta(2, 2)
    wait_meta(0, 0)
    pltpu.async_copy(x_hbm.at[srcb[0]], eb[0], gsem[0], add=True)

    def outer(t, carry):
        for k in range(NBUF):
            slot(t * NBUF + k, k)
        return carry

    lax.fori_loop(0, NSLOTS // NBUF, outer, 0)

    plsc.subcore_barrier()
    pltpu.sync_copy(acc.at[pl.ds(sid * ROWS_PER_TILE, ROWS_PER_TILE)],
                    out_hbm.at[cid, pl.ds(sid * ROWS_PER_TILE, ROWS_PER_TILE)])

    @pl.when(sid == NS - 1)
    def _write_tail():
        pltpu.sync_copy(acc.at[pl.ds(NS * ROWS_PER_TILE, TAIL_ROWS)],
                        out_hbm.at[cid, pl.ds(NS * ROWS_PER_TILE, TAIL_ROWS)])


_sc_layer = functools.partial(
    pl.kernel,
    out_type=jax.ShapeDtypeStruct((NC, N, D), jnp.float32),
    mesh=plsc.VectorSubcoreMesh(core_axis_name="c", subcore_axis_name="s"),
    scratch_types=(
        [pltpu.VMEM((CHUNK,), jnp.int32) for _ in range(NBUF)]
        + [pltpu.VMEM((CHUNK,), jnp.int32) for _ in range(NBUF)]
        + [pltpu.VMEM((CHUNK, D), jnp.float32) for _ in range(NBUF)]
        + [pltpu.VMEM_SHARED((N, D), jnp.float32)]
        + [pltpu.SemaphoreType.DMA for _ in range(3 * NBUF)]
    ),
)(_sc_body)


_EBLK = 8000


def _tc_edge_body(attr_ref, we_ref, be_ref, o_ref):
    o_ref[...] = (jnp.dot(attr_ref[...], we_ref[...],
                          preferred_element_type=jnp.float32) + be_ref[...])


_tc_edge = pl.pallas_call(
    _tc_edge_body,
    grid=(E // _EBLK,),
    in_specs=[pl.BlockSpec((_EBLK, ED), lambda i: (i, 0)),
              pl.BlockSpec((ED, D), lambda i: (0, 0)),
              pl.BlockSpec((1, D), lambda i: (0, 0))],
    out_specs=pl.BlockSpec((_EBLK, D), lambda i: (i, 0)),
    out_shape=jax.ShapeDtypeStruct((E, D), jnp.float32),
)


def _tc_body_mid(x_ref, acc_ref, wm_ref, bm_ref, o_ref):
    h = x_ref[...] + acc_ref[0] + acc_ref[1]
    h = jnp.maximum(jnp.dot(h, wm_ref[0], preferred_element_type=jnp.float32)
                    + bm_ref[0], 0.0)
    h = jnp.maximum(jnp.dot(h, wm_ref[1], preferred_element_type=jnp.float32)
                    + bm_ref[1], 0.0)
    h = jnp.dot(h, wm_ref[2], preferred_element_type=jnp.float32) + bm_ref[2]
    o_ref[...] = jnp.maximum(h, 0.0) + x_ref[...]


def _tc_body_final(x_ref, acc_ref, wm_ref, bm_ref, wp1_ref, bp1_ref, wp2_ref,
                   bp2_ref, o_ref):
    h = x_ref[...] + acc_ref[0] + acc_ref[1]
    h = jnp.maximum(jnp.dot(h, wm_ref[0], preferred_element_type=jnp.float32)
                    + bm_ref[0], 0.0)
    h = jnp.maximum(jnp.dot(h, wm_ref[1], preferred_element_type=jnp.float32)
                    + bm_ref[1], 0.0)
    h = jnp.dot(h, wm_ref[2], preferred_element_type=jnp.float32) + bm_ref[2]
    h = jnp.maximum(h, 0.0) + x_ref[...]
    h = jnp.maximum(jnp.dot(h, wp1_ref[...], preferred_element_type=jnp.float32)
                    + bp1_ref[...], 0.0)
    o_ref[...] = (jnp.dot(h, wp2_ref[...], preferred_element_type=jnp.float32)
                  + bp2_ref[...])


_TC_BLK = 1000
_TC_GRID = N // _TC_BLK

_x_spec = pl.BlockSpec((_TC_BLK, D), lambda i: (i, 0))
_acc_spec = pl.BlockSpec((NC, _TC_BLK, D), lambda i: (0, i, 0))
_wm_spec = pl.BlockSpec((3, D, D), lambda i: (0, 0, 0))
_bm_spec = pl.BlockSpec((3, D), lambda i: (0, 0))
_w_spec = pl.BlockSpec((D, D), lambda i: (0, 0))
_b_spec = pl.BlockSpec((1, D), lambda i: (0, 0))

_tc_mid = pl.pallas_call(
    _tc_body_mid,
    grid=(_TC_GRID,),
    in_specs=[_x_spec, _acc_spec, _wm_spec, _bm_spec],
    out_specs=_x_spec,
    out_shape=jax.ShapeDtypeStruct((N, D), jnp.float32),
)

_tc_final = pl.pallas_call(
    _tc_body_final,
    grid=(_TC_GRID,),
    in_specs=[_x_spec, _acc_spec, _wm_spec, _bm_spec,
              _w_spec, _b_spec, _w_spec, _b_spec],
    out_specs=_x_spec,
    out_shape=jax.ShapeDtypeStruct((N, D), jnp.float32),
)


def kernel(x, edge_index, edge_attr, We, be, Wm, bm, Wp1, bp1, Wp2, bp2):
    src = edge_index[0]
    dst = edge_index[1]
    zeros_nd = jnp.zeros((N, D), jnp.float32)
    bp1r = bp1.reshape(1, D)
    bp2r = bp2.reshape(1, D)
    els = [_tc_edge(edge_attr, We[l], be[l].reshape(1, D)) for l in range(3)]
    for l in range(3):
        agg = _sc_layer(src, dst, els[l], zeros_nd, x)
        if l < 2:
            x = _tc_mid(x, agg, Wm[l], bm[l])
        else:
            x = _tc_final(x, agg, Wm[l], bm[l], Wp1, bp1r, Wp2, bp2r)
    return x


# speedup vs baseline: 4.8093x; 1.1549x over previous
import functools

import jax
import jax.numpy as jnp
from jax import lax
from jax.experimental import pallas as pl
from jax.experimental.pallas import tpu as pltpu
from jax.experimental.pallas import tpu_sc as plsc

N, E, D, ED = 10000, 320000, 128, 3
NC, NS, LANES = 2, 16, 16
NW = NC * NS
EW = E // NW
CHUNK = 80
NCHUNKS = EW // CHUNK
NBUF = 4
NSLOTS = ((NCHUNKS + NBUF - 1) // NBUF) * NBUF
ROWS_PER_TILE = 624
TAIL_ROWS = N - NS * ROWS_PER_TILE
DC = D // LANES


def _sc_body(src_hbm, dst_hbm, el_hbm, zeros_hbm, x_hbm, out_hbm, *scr):
    srcb = scr[0:NBUF]
    dstb = scr[NBUF:2 * NBUF]
    eb = scr[2 * NBUF:3 * NBUF]
    acc = scr[3 * NBUF]
    msem = scr[3 * NBUF + 1:4 * NBUF + 1]
    gsem = scr[4 * NBUF + 1:5 * NBUF + 1]
    ssem = scr[5 * NBUF + 1:6 * NBUF + 1]

    cid = lax.axis_index("c")
    sid = lax.axis_index("s")
    wid = sid * NC + cid
    base = wid * EW

    pltpu.sync_copy(zeros_hbm.at[pl.ds(sid * ROWS_PER_TILE, ROWS_PER_TILE)],
                    acc.at[pl.ds(sid * ROWS_PER_TILE, ROWS_PER_TILE)])

    @pl.when(sid == NS - 1)
    def _zero_tail():
        pltpu.sync_copy(zeros_hbm.at[pl.ds(NS * ROWS_PER_TILE, TAIL_ROWS)],
                        acc.at[pl.ds(NS * ROWS_PER_TILE, TAIL_ROWS)])

    plsc.subcore_barrier()

    def meta_descs(c, b):
        off = base + c * CHUNK
        return (
            pltpu.make_async_copy(src_hbm.at[pl.ds(off, CHUNK)], srcb[b],
                                  msem[b]),
            pltpu.make_async_copy(dst_hbm.at[pl.ds(off, CHUNK)], dstb[b],
                                  msem[b]),
            pltpu.make_async_copy(el_hbm.at[pl.ds(off, CHUNK)], eb[b],
                                  msem[b]),
        )

    def issue_meta(c, b):
        for d in meta_descs(c, b):
            d.start()

    def wait_meta(c, b):
        for d in meta_descs(c, b):
            d.wait()

    def gather_desc(b):
        return pltpu.make_async_copy(x_hbm.at[srcb[b]], eb[b], gsem[b])

    def scatter_desc(b):
        return pltpu.make_async_copy(eb[b], acc.at[dstb[b]], ssem[b])

    def compute(b):
        ebuf = eb[b]

        @plsc.parallel_loop(0, CHUNK, 1, unroll=4)
        def edge_body(e):
            for c in range(DC):
                v = ebuf[e, pl.ds(c * LANES, LANES)]
                ebuf[e, pl.ds(c * LANES, LANES)] = jnp.maximum(v, 0.0)

    def slot(s, k):
        @pl.when(s <= NCHUNKS - 1)
        def _wait_gather():
            gather_desc(k).wait()

        @pl.when(jnp.logical_and(s >= 2, s - 2 <= NCHUNKS - 1))
        def _drain_scatter():
            scatter_desc((k - 2) % NBUF).wait()

        @pl.when(s + 2 <= NCHUNKS - 1)
        def _prefetch_meta():
            issue_meta(s + 2, (k + 2) % NBUF)

        @pl.when(s + 1 <= NCHUNKS - 1)
        def _launch_gather():
            wait_meta(s + 1, (k + 1) % NBUF)
            pltpu.async_copy(x_hbm.at[srcb[(k + 1) % NBUF]],
                             eb[(k + 1) % NBUF], gsem[(k + 1) % NBUF],
                             add=True)

        @pl.when(s <= NCHUNKS - 1)
        def _compute_scatter():
            compute(k)
            pltpu.async_copy(eb[k], acc.at[dstb[k]], ssem[k], add=True)

    issue_meta(0, 0)
    issue_meta(1, 1)
    wait_meta(0, 0)
    pltpu.async_copy(x_hbm.at[srcb[0]], eb[0], gsem[0], add=True)

    def outer(t, carry):
        for k in range(NBUF):
            slot(t * NBUF + k, k)
        return carry

    lax.fori_loop(0, NSLOTS // NBUF, outer, 0)

    plsc.subcore_barrier()
    pltpu.sync_copy(acc.at[pl.ds(sid * ROWS_PER_TILE, ROWS_PER_TILE)],
                    out_hbm.at[cid, pl.ds(sid * ROWS_PER_TILE, ROWS_PER_TILE)])

    @pl.when(sid == NS - 1)
    def _write_tail():
        pltpu.sync_copy(acc.at[pl.ds(NS * ROWS_PER_TILE, TAIL_ROWS)],
                        out_hbm.at[cid, pl.ds(NS * ROWS_PER_TILE, TAIL_ROWS)])


_sc_layer = functools.partial(
    pl.kernel,
    out_type=jax.ShapeDtypeStruct((NC, N, D), jnp.float32),
    mesh=plsc.VectorSubcoreMesh(core_axis_name="c", subcore_axis_name="s"),
    scratch_types=(
        [pltpu.VMEM((CHUNK,), jnp.int32) for _ in range(NBUF)]
        + [pltpu.VMEM((CHUNK,), jnp.int32) for _ in range(NBUF)]
        + [pltpu.VMEM((CHUNK, D), jnp.float32) for _ in range(NBUF)]
        + [pltpu.VMEM_SHARED((N, D), jnp.float32)]
        + [pltpu.SemaphoreType.DMA for _ in range(3 * NBUF)]
    ),
)(_sc_body)


_EBLK = 8000


def _tc_edge_body(attr_ref, we_ref, be_ref, o_ref):
    o_ref[...] = (jnp.dot(attr_ref[...], we_ref[...],
                          preferred_element_type=jnp.float32) + be_ref[...])


_tc_edge = pl.pallas_call(
    _tc_edge_body,
    grid=(E // _EBLK,),
    in_specs=[pl.BlockSpec((_EBLK, ED), lambda i: (i, 0)),
              pl.BlockSpec((ED, D), lambda i: (0, 0)),
              pl.BlockSpec((1, D), lambda i: (0, 0))],
    out_specs=pl.BlockSpec((_EBLK, D), lambda i: (i, 0)),
    out_shape=jax.ShapeDtypeStruct((E, D), jnp.float32),
)


def _tc_body_mid(x_ref, acc_ref, wm_ref, bm_ref, o_ref):
    h = x_ref[...] + acc_ref[0] + acc_ref[1]
    h = jnp.maximum(jnp.dot(h, wm_ref[0], preferred_element_type=jnp.float32)
                    + bm_ref[0], 0.0)
    h = jnp.maximum(jnp.dot(h, wm_ref[1], preferred_element_type=jnp.float32)
                    + bm_ref[1], 0.0)
    h = jnp.dot(h, wm_ref[2], preferred_element_type=jnp.float32) + bm_ref[2]
    o_ref[...] = jnp.maximum(h, 0.0) + x_ref[...]


def _tc_body_final(x_ref, acc_ref, wm_ref, bm_ref, wp1_ref, bp1_ref, wp2_ref,
                   bp2_ref, o_ref):
    h = x_ref[...] + acc_ref[0] + acc_ref[1]
    h = jnp.maximum(jnp.dot(h, wm_ref[0], preferred_element_type=jnp.float32)
                    + bm_ref[0], 0.0)
    h = jnp.maximum(jnp.dot(h, wm_ref[1], preferred_element_type=jnp.float32)
                    + bm_ref[1], 0.0)
    h = jnp.dot(h, wm_ref[2], preferred_element_type=jnp.float32) + bm_ref[2]
    h = jnp.maximum(h, 0.0) + x_ref[...]
    h = jnp.maximum(jnp.dot(h, wp1_ref[...], preferred_element_type=jnp.float32)
                    + bp1_ref[...], 0.0)
    o_ref[...] = (jnp.dot(h, wp2_ref[...], preferred_element_type=jnp.float32)
                  + bp2_ref[...])


_TC_BLK = 1000
_TC_GRID = N // _TC_BLK

_x_spec = pl.BlockSpec((_TC_BLK, D), lambda i: (i, 0))
_acc_spec = pl.BlockSpec((NC, _TC_BLK, D), lambda i: (0, i, 0))
_wm_spec = pl.BlockSpec((3, D, D), lambda i: (0, 0, 0))
_bm_spec = pl.BlockSpec((3, D), lambda i: (0, 0))
_w_spec = pl.BlockSpec((D, D), lambda i: (0, 0))
_b_spec = pl.BlockSpec((1, D), lambda i: (0, 0))

_tc_mid = pl.pallas_call(
    _tc_body_mid,
    grid=(_TC_GRID,),
    in_specs=[_x_spec, _acc_spec, _wm_spec, _bm_spec],
    out_specs=_x_spec,
    out_shape=jax.ShapeDtypeStruct((N, D), jnp.float32),
)

_tc_final = pl.pallas_call(
    _tc_body_final,
    grid=(_TC_GRID,),
    in_specs=[_x_spec, _acc_spec, _wm_spec, _bm_spec,
              _w_spec, _b_spec, _w_spec, _b_spec],
    out_specs=_x_spec,
    out_shape=jax.ShapeDtypeStruct((N, D), jnp.float32),
)


def kernel(x, edge_index, edge_attr, We, be, Wm, bm, Wp1, bp1, Wp2, bp2):
    src = edge_index[0]
    dst = edge_index[1]
    zeros_nd = jnp.zeros((N, D), jnp.float32)
    bp1r = bp1.reshape(1, D)
    bp2r = bp2.reshape(1, D)
    els = [_tc_edge(edge_attr, We[l], be[l].reshape(1, D)) for l in range(3)]
    for l in range(3):
        agg = _sc_layer(src, dst, els[l], zeros_nd, x)
        if l < 2:
            x = _tc_mid(x, agg, Wm[l], bm[l])
        else:
            x = _tc_final(x, agg, Wm[l], bm[l], Wp1, bp1r, Wp2, bp2r)
    return x
